# baseline (speedup 1.0000x reference)
import functools

import jax
import jax.numpy as jnp
from jax import lax
from jax.experimental import pallas as pl
from jax.experimental.pallas import tpu as pltpu
from jax.experimental.pallas import tpu_sc as plsc

VOCAB = 1000000
DIM = 32
BATCH = 16384

NUM_CORES = 2
NUM_SUBCORES = 16
LANES = 16
NW = NUM_CORES * NUM_SUBCORES
BPW = BATCH // NW
GROUPS = BPW // LANES

_DNUMS = jax.lax.GatherDimensionNumbers(
    offset_dims=(), collapsed_slice_dims=(0,), start_index_map=(0,))


def _permute(v, idx):
    return jax.lax.gather(
        v, idx.reshape(LANES, 1), _DNUMS, slice_sizes=(1,),
        mode=jax.lax.GatherScatterMode.PROMISE_IN_BOUNDS)


def _sc_kernel_body(word_hbm, ctx_hbm, ht_hbm, ct_hbm, out_hbm,
                    wi_v, ci_v, wr_v, cr_v, o_v, sem_w, sem_c):
    wid = lax.axis_index("s") * NUM_CORES + lax.axis_index("c")
    base = wid * BPW

    pltpu.sync_copy(word_hbm.at[pl.ds(base, BPW)], wi_v)
    pltpu.sync_copy(ctx_hbm.at[pl.ds(base, BPW)], ci_v)
    cp_w = pltpu.async_copy(ht_hbm.at[wi_v], wr_v, sem_w)
    cp_c = pltpu.async_copy(ct_hbm.at[ci_v], cr_v, sem_c)
    cp_w.wait()
    cp_c.wait()

    def group_body(g, _):
        row0 = g * LANES
        acc = jnp.zeros((LANES,), jnp.float32)
        for i in range(LANES):
            r = row0 + i
            p = (wr_v[r, pl.ds(0, LANES)] * cr_v[r, pl.ds(0, LANES)]
                 + wr_v[r, pl.ds(LANES, LANES)] * cr_v[r, pl.ds(LANES, LANES)])
            for sh in (8, 4, 2, 1):
                p = p + _permute(p, jnp.bitwise_xor(lax.iota(jnp.int32, LANES), sh))
            acc = jnp.where(lax.iota(jnp.int32, LANES) == i, p, acc)
        o_v[pl.ds(row0, LANES)] = 1.0 / (1.0 + jnp.exp(-acc))
        return 0

    lax.fori_loop(0, GROUPS, group_body, 0)
    pltpu.sync_copy(o_v, out_hbm.at[pl.ds(base, BPW)])


_sc_call = functools.partial(
    pl.kernel,
    out_type=jax.ShapeDtypeStruct((BATCH,), jnp.float32),
    mesh=plsc.VectorSubcoreMesh(
        core_axis_name="c", subcore_axis_name="s",
        num_cores=NUM_CORES, num_subcores=NUM_SUBCORES),
    compiler_params=pltpu.CompilerParams(use_tc_tiling_on_sc=False),
    scratch_types=[
        pltpu.VMEM((BPW,), jnp.int32),
        pltpu.VMEM((BPW,), jnp.int32),
        pltpu.VMEM((BPW, DIM), jnp.float32),
        pltpu.VMEM((BPW, DIM), jnp.float32),
        pltpu.VMEM((BPW,), jnp.float32),
        pltpu.SemaphoreType.DMA,
        pltpu.SemaphoreType.DMA,
    ],
)(_sc_kernel_body)


def kernel(word, context, hidden_table, context_table):
    out = _sc_call(word.astype(jnp.int32), context.astype(jnp.int32),
                   hidden_table, context_table)
    return out.reshape(BATCH, 1)
